# baseline (speedup 1.0000x reference)
import functools

import jax
import jax.numpy as jnp
from jax import lax
from jax.experimental import pallas as pl
from jax.experimental.pallas import tpu as pltpu
from jax.experimental.pallas import tpu_sc as plsc

VOCAB = 1000000
HIDDEN = 64
NUM_CLASSES = 3
BATCH = 4096
SEQ = 200

NC = 2
NS = 16
NW = NC * NS
PCOLS = 8
ROWS_PER_W = BATCH // NW
TOKS_PER_W = ROWS_PER_W * SEQ
GRP = 8
CH_TOK = GRP * SEQ
NCHUNK = ROWS_PER_W // GRP
LIST_SPANS = [(k * 128, 128) for k in range(CH_TOK // 128)] + (
    [(CH_TOK - CH_TOK % 128, CH_TOK % 128)] if CH_TOK % 128 else [])
NSLOT = 2

PBLK = 10000


def _proj_kernel(x_ref, w_ref, o_ref):
  o_ref[...] = jnp.dot(x_ref[...], w_ref[...],
                       preferred_element_type=jnp.float32)


def _sc_kernel(ids_hbm, proj_hbm, b_hbm, out_hbm,
               idx_v, gslot_v, b_v, out_v, *sems):
  wid = lax.axis_index("s") * NC + lax.axis_index("c")
  tok_base = wid * TOKS_PER_W

  pltpu.sync_copy(ids_hbm.at[pl.ds(tok_base, TOKS_PER_W)], idx_v)
  pltpu.sync_copy(b_hbm, b_v)

  io16 = lax.iota(jnp.int32, 16)
  bvec = b_v[pl.ds(0, 16)]
  cidx = [io16 * 0 + cc for cc in range(NUM_CLASSES)]
  tail_mask = io16 < 8
  zero16 = jnp.zeros((16,), jnp.float32)

  def fire(b, g):
    for off, ln in LIST_SPANS:
      pltpu.async_copy(
          proj_hbm.at[idx_v.at[pl.ds(g * CH_TOK + off, ln)]],
          gslot_v.at[pl.ds(b * CH_TOK + off, ln)],
          sems[b])

  def wait_slot(b):
    for off, ln in LIST_SPANS:
      pltpu.make_async_copy(proj_hbm.at[pl.ds(0, ln)],
                            gslot_v.at[pl.ds(b * CH_TOK + off, ln)],
                            sems[b]).wait()

  fire(0, 0)

  def process(b, g):
    wait_slot(b)

    def row_body(r, _):
      tbase = b * CH_TOK + r * SEQ
      logits = []
      for cc in range(NUM_CLASSES):
        acc = zero16
        for j in range(12):
          ridx = io16 + (tbase + 16 * j)
          acc = acc + plsc.load_gather(gslot_v, [ridx, cidx[cc]])
        ridx = io16 + (tbase + 192)
        lg = plsc.load_gather(gslot_v, [ridx, cidx[cc]], mask=tail_mask)
        acc = acc + jnp.where(tail_mask, lg, zero16)
        logits.append(jnp.sum(acc) + bvec[cc])
      row = g * GRP + r
      lv = jnp.where(io16 == 0, logits[0],
                     jnp.where(io16 == 1, logits[1], logits[2]))
      plsc.store_scatter(out_v, [NUM_CLASSES * row + io16], lv,
                         mask=io16 < NUM_CLASSES)
      return 0

    lax.fori_loop(0, GRP, row_body, 0)

  def outer(j2, _):
    for b in range(NSLOT):
      g = j2 * NSLOT + b

      @pl.when(g + 1 < NCHUNK)
      def _():
        fire((b + 1) % NSLOT, g + 1)

      process(b, g)
    return 0

  lax.fori_loop(0, NCHUNK // NSLOT, outer, 0)

  obase = wid * ROWS_PER_W * NUM_CLASSES
  pltpu.sync_copy(out_v, out_hbm.at[pl.ds(obase, ROWS_PER_W * NUM_CLASSES)])


@jax.jit
def _run(ids_flat, table, wp, bpad):
  proj = pl.pallas_call(
      _proj_kernel,
      grid=(VOCAB // PBLK,),
      in_specs=[
          pl.BlockSpec((PBLK, HIDDEN), lambda i: (i, 0)),
          pl.BlockSpec((HIDDEN, PCOLS), lambda i: (0, 0)),
      ],
      out_specs=pl.BlockSpec((PBLK, PCOLS), lambda i: (i, 0)),
      out_shape=jax.ShapeDtypeStruct((VOCAB, PCOLS), jnp.float32),
  )(table, wp)

  mesh = plsc.VectorSubcoreMesh(core_axis_name="c", subcore_axis_name="s")
  out = pl.kernel(
      _sc_kernel,
      out_type=jax.ShapeDtypeStruct((BATCH * NUM_CLASSES,), jnp.float32),
      mesh=mesh,
      scratch_types=[
          pltpu.VMEM((TOKS_PER_W,), jnp.int32),
          pltpu.VMEM((NSLOT * CH_TOK, PCOLS), jnp.float32),
          pltpu.VMEM((16,), jnp.float32),
          pltpu.VMEM((ROWS_PER_W * NUM_CLASSES,), jnp.float32),
          pltpu.SemaphoreType.DMA,
          pltpu.SemaphoreType.DMA,
      ],
      compiler_params=pltpu.CompilerParams(needs_layout_passes=False,
                                           use_tc_tiling_on_sc=False),
  )(ids_flat, proj, bpad)
  return out


def kernel(input_ids, emb_table, W, b):
  ids_flat = input_ids.astype(jnp.int32).reshape(BATCH * SEQ)
  wp = jnp.concatenate(
      [W.astype(jnp.float32) * jnp.float32(1.0 / SEQ),
       jnp.zeros((HIDDEN, PCOLS - NUM_CLASSES), jnp.float32)], axis=1)
  bpad = jnp.pad(b.astype(jnp.float32), (0, 16 - NUM_CLASSES))
  return _run(ids_flat, emb_table, wp, bpad).reshape(BATCH, NUM_CLASSES)


# speedup vs baseline: 1.5242x; 1.5242x over previous
import functools

import jax
import jax.numpy as jnp
from jax import lax
from jax.experimental import pallas as pl
from jax.experimental.pallas import tpu as pltpu
from jax.experimental.pallas import tpu_sc as plsc

VOCAB = 1000000
HIDDEN = 64
NUM_CLASSES = 3
BATCH = 4096
SEQ = 200

NC = 2
NS = 16
NW = NC * NS
ROWS_PER_W = BATCH // NW
HALF = SEQ // 2
IDXROWS_PER_W = 2 * ROWS_PER_W

PAIRS_PER_W = ROWS_PER_W // 2
STREAMS_PER_PAIR = 4
PAIR_ROWS = 2 * SEQ
NSLOT = 4


def _sc_kernel(ids_hbm, table_hbm, wt_hbm, b_hbm, out_hbm,
               idx_v, rows_v, w_v, b_v, out_v, *sems):
  wid = lax.axis_index("s") * NC + lax.axis_index("c")
  ibase = wid * IDXROWS_PER_W
  obase = wid * ROWS_PER_W

  pltpu.sync_copy(ids_hbm.at[pl.ds(ibase, IDXROWS_PER_W)], idx_v)
  pltpu.sync_copy(wt_hbm, w_v)
  pltpu.sync_copy(b_hbm, b_v)

  inv = jnp.float32(1.0 / SEQ)
  bvec = b_v[pl.ds(0, 16)]
  iot = lax.iota(jnp.int32, 16)

  def fire(slot, p):
    for k in range(STREAMS_PER_PAIR):
      pltpu.async_copy(table_hbm.at[idx_v.at[STREAMS_PER_PAIR * p + k]],
                       rows_v.at[pl.ds(slot * PAIR_ROWS + k * HALF, HALF)],
                       sems[slot])

  def wait_slot(slot):
    pltpu.make_async_copy(table_hbm.at[pl.ds(0, PAIR_ROWS)],
                          rows_v.at[pl.ds(slot * PAIR_ROWS, PAIR_ROWS)],
                          sems[slot]).wait()

  for p in range(NSLOT - 1):
    fire(p, p)

  zero = jnp.zeros((16,), jnp.float32)

  def emit(i, a):
    a0, a1, a2, a3 = a
    logits = []
    for c in range(NUM_CLASSES):
      s = (jnp.sum(a0 * w_v[c, pl.ds(0, 16)]) +
           jnp.sum(a1 * w_v[c, pl.ds(16, 16)]) +
           jnp.sum(a2 * w_v[c, pl.ds(32, 16)]) +
           jnp.sum(a3 * w_v[c, pl.ds(48, 16)]))
      logits.append(s * inv + bvec[c])
    lv = jnp.where(iot == 0, logits[0],
                   jnp.where(iot == 1, logits[1], logits[2]))
    plsc.store_scatter(out_v, [NUM_CLASSES * i + iot], lv,
                       mask=iot < NUM_CLASSES)

  def outer(j, _):
    for b in range(NSLOT):
      p = j * NSLOT + b

      @pl.when(p + NSLOT - 1 < PAIRS_PER_W)
      def _():
        fire((b + NSLOT - 1) % NSLOT, p + NSLOT - 1)

      wait_slot(b)

      for half in range(2):
        base = b * PAIR_ROWS + half * SEQ

        def acc_body(k, a):
          a0, a1, a2, a3 = a
          a0 = a0 + rows_v[base + k, pl.ds(0, 16)]
          a1 = a1 + rows_v[base + k, pl.ds(16, 16)]
          a2 = a2 + rows_v[base + k, pl.ds(32, 16)]
          a3 = a3 + rows_v[base + k, pl.ds(48, 16)]
          return (a0, a1, a2, a3)

        acc = lax.fori_loop(0, SEQ, acc_body,
                            (zero, zero, zero, zero), unroll=10)
        emit(2 * p + half, acc)
    return 0

  lax.fori_loop(0, PAIRS_PER_W // NSLOT, outer, 0)

  pltpu.sync_copy(out_v, out_hbm.at[pl.ds(obase * NUM_CLASSES,
                                          ROWS_PER_W * NUM_CLASSES)])


@jax.jit
def _run(ids2, table, wt, bpad):
  mesh = plsc.VectorSubcoreMesh(core_axis_name="c", subcore_axis_name="s")
  f = functools.partial(
      pl.kernel,
      out_type=jax.ShapeDtypeStruct((BATCH * NUM_CLASSES,), jnp.float32),
      mesh=mesh,
      scratch_types=[
          pltpu.VMEM((IDXROWS_PER_W, HALF), jnp.int32),
          pltpu.VMEM((NSLOT * PAIR_ROWS, HIDDEN), jnp.float32),
          pltpu.VMEM((NUM_CLASSES, HIDDEN), jnp.float32),
          pltpu.VMEM((16,), jnp.float32),
          pltpu.VMEM((ROWS_PER_W * NUM_CLASSES,), jnp.float32),
      ] + [pltpu.SemaphoreType.DMA] * NSLOT,
      compiler_params=pltpu.CompilerParams(needs_layout_passes=False,
                                           use_tc_tiling_on_sc=False),
  )(_sc_kernel)
  return f(ids2, table, wt, bpad)


def kernel(input_ids, emb_table, W, b):
  ids2 = input_ids.astype(jnp.int32).reshape(BATCH * SEQ // HALF, HALF)
  wt = W.T
  bpad = jnp.pad(b.astype(jnp.float32), (0, 16 - NUM_CLASSES))
  return _run(ids2, emb_table, wt, bpad).reshape(BATCH, NUM_CLASSES)
